# baseline (speedup 1.0000x reference)
import functools

import jax
import jax.numpy as jnp
from jax import lax
from jax.experimental import pallas as pl
from jax.experimental.pallas import tpu as pltpu
from jax.experimental.pallas import tpu_sc as plsc

BATCH = 16384
HIST_LEN = 50
EMBED_DIM = 64
VOCAB = 1000000

_info = plsc.get_sparse_core_info()
NC, NS = _info.num_cores, _info.num_subcores
NW = NC * NS
BPW = BATCH // NW
CB = BPW // 128
NBUF = CB


def _body(x_hbm, w_hbm, out_hbm, xs, cidx, gbuf, tbuf, gsems, ssems):
    wid = lax.axis_index("s") * NC + lax.axis_index("c")
    pltpu.sync_copy(x_hbm.at[pl.ds(pl.multiple_of(wid * (BPW * HIST_LEN), 8), BPW * HIST_LEN)], xs)

    iota = lax.iota(jnp.int32, 16)

    def build_cidx(t, b):
        for v in range(8):
            flat = (iota + (128 * b + 16 * v)) * HIST_LEN + t
            vals = plsc.load_gather(xs, [flat])
            cidx[b, pl.ds(16 * v, 16)] = vals

    def gather(b):
        return pltpu.make_async_copy(w_hbm.at[cidx.at[b]], gbuf.at[b], gsems.at[b])

    def store(t, b):
        cbg = wid * CB + b
        return pltpu.make_async_copy(tbuf.at[b], out_hbm.at[t, :, cbg], ssems.at[b])

    def transpose(b):
        def row(d, carry):
            e = d // 8
            f = lax.rem(d, 8)
            cols = jnp.full((16,), d, jnp.int32)
            for v in range(8):
                vals = plsc.load_gather(gbuf.at[b], [iota + 16 * v, cols])
                tbuf[b, e, f, pl.ds(16 * v, 16)] = vals
            return carry
        lax.fori_loop(0, EMBED_DIM, row, 0)

    for b in range(NBUF):
        build_cidx(0, b)
        gather(b).start()

    def t_step(t, carry):
        for b in range(NBUF):
            gather(b).wait()

            @pl.when(t > 0)
            def _():
                store(t - 1, b).wait()

            transpose(b)
            store(t, b).start()

            @pl.when(t + 1 < HIST_LEN)
            def _():
                build_cidx(t + 1, b)
                gather(b).start()
        return carry

    lax.fori_loop(0, HIST_LEN, t_step, 0)
    for b in range(NBUF):
        store(HIST_LEN - 1, b).wait()


@jax.jit
def _embed(x, wpad):
    mesh = plsc.VectorSubcoreMesh(core_axis_name="c", subcore_axis_name="s")
    return pl.kernel(
        _body,
        mesh=mesh,
        out_type=jax.ShapeDtypeStruct((HIST_LEN, 8, BATCH // 128, 8, 128), jnp.float32),
        scratch_types=[
            pltpu.VMEM((BPW * HIST_LEN,), jnp.int32),
            pltpu.VMEM((NBUF, 128), jnp.int32),
            pltpu.VMEM((NBUF, 128, 128), jnp.float32),
            pltpu.VMEM((NBUF, 8, 8, 128), jnp.float32),
            pltpu.SemaphoreType.DMA((NBUF,)),
            pltpu.SemaphoreType.DMA((NBUF,)),
        ],
        compiler_params=pltpu.CompilerParams(
            use_tc_tiling_on_sc=False, needs_layout_passes=False),
    )(x.reshape(BATCH * HIST_LEN), wpad)


def kernel(x, weight):
    wpad = jnp.pad(weight, ((0, 0), (0, EMBED_DIM)))
    p = _embed(x.astype(jnp.int32), wpad)
    return p.transpose(2, 4, 0, 1, 3).reshape(BATCH, HIST_LEN, EMBED_DIM)
